# baseline (speedup 1.0000x reference)
import functools

import jax
import jax.numpy as jnp
from jax import lax
from jax.experimental import pallas as pl
from jax.experimental.pallas import tpu as pltpu
from jax.experimental.pallas import tpu_sc as plsc

N = 10000
E = 160000
D_IN = 256
D_HID = 128
D_OUT = 256

NC = 2
NS = 16
NW = NC * NS
E_PER = E // NW
CH = 40
ITERS = E_PER // CH
N_PAD = 10240
ROWS_PER_TILE = N_PAD // NS
CP = 40
NCP = ROWS_PER_TILE // CP
DP = 128
NDP = ROWS_PER_TILE // DP


def _sc_segsum():
    out_type = [jax.ShapeDtypeStruct((N_PAD, D_HID), jnp.float32),
                jax.ShapeDtypeStruct((N_PAD, D_HID), jnp.float32)]

    scratch = [
        pltpu.VMEM_SHARED((N_PAD, D_HID), jnp.float32),
        pltpu.VMEM((CP, D_HID), jnp.float32),
        pltpu.VMEM((CP, D_HID), jnp.float32),
        pltpu.VMEM((CH, D_HID), jnp.float32),
        pltpu.VMEM((CH, D_HID), jnp.float32),
        pltpu.VMEM((E_PER,), jnp.int32),
        pltpu.VMEM((E_PER,), jnp.int32),
        pltpu.SemaphoreType.DMA,
        pltpu.SemaphoreType.DMA,
        pltpu.SemaphoreType.DMA,
    ]

    mesh = plsc.VectorSubcoreMesh(core_axis_name="c", subcore_axis_name="s",
                                  num_cores=NC, num_subcores=NS)

    def body(tab, eidx, zrows, acc0, acc1, acc_sh, vbuf0, vbuf1,
             rows0, rows1, sbuf, dstbuf, gsem, ssem, osem):
        cid = lax.axis_index("c")
        sid = lax.axis_index("s")
        wid = cid * NS + sid
        rows = (rows0, rows1)
        vbufs = (vbuf0, vbuf1)
        row0 = sid * ROWS_PER_TILE

        base = wid * E_PER
        ih1 = pltpu.async_copy(eidx.at[0, pl.ds(base, E_PER)], sbuf, osem)
        ih2 = pltpu.async_copy(eidx.at[1, pl.ds(base, E_PER)], dstbuf, osem)

        pltpu.sync_copy(zrows, vbuf0)
        zh = [pltpu.async_copy(vbuf0, acc_sh.at[pl.ds(row0 + k * CP, CP)],
                               gsem)
              for k in range(NCP)]
        for h in zh:
            h.wait()
        ih1.wait()
        ih2.wait()
        plsc.subcore_barrier()

        def gather_start(i, b):
            pltpu.async_copy(tab.at[sbuf.at[pl.ds(i * CH, CH)]], rows[b], gsem)

        def gather_wait(i, b):
            pltpu.make_async_copy(tab.at[sbuf.at[pl.ds(i * CH, CH)]], rows[b],
                                  gsem).wait()

        def process(i, b):
            gather_wait(i, b)

            @pl.when(i >= 1)
            def _():
                pltpu.make_async_copy(
                    rows[1 - b],
                    acc_sh.at[dstbuf.at[pl.ds((i - 1) * CH, CH)]],
                    ssem).wait()

            @pl.when(i < ITERS - 1)
            def _():
                gather_start(i + 1, 1 - b)

            pltpu.make_async_copy(rows[b],
                                  acc_sh.at[dstbuf.at[pl.ds(i * CH, CH)]],
                                  ssem).start(add=True)

        gather_start(0, 0)

        def body2(k, _):
            process(2 * k, 0)
            process(2 * k + 1, 1)
            return 0

        lax.fori_loop(0, ITERS // 2, body2, 0)
        for i in range(ITERS - ITERS % 2, ITERS):
            process(i, i % 2)
        pltpu.make_async_copy(
            rows[(ITERS - 1) % 2],
            acc_sh.at[dstbuf.at[pl.ds((ITERS - 1) * CH, CH)]], ssem).wait()
        plsc.subcore_barrier()

        def out_desc(k, p):
            return pltpu.make_async_copy(
                vbufs[p], acc0.at[pl.ds(row0 + k * CP, CP)], osem)

        for k in range(NCP):
            p = k % 2
            if k >= 2:
                out_desc(k - 2, p).wait()
            pltpu.sync_copy(acc_sh.at[pl.ds(row0 + k * CP, CP)], vbufs[p])

            @pl.when(cid == 0)
            def _():
                pltpu.make_async_copy(
                    vbufs[p], acc0.at[pl.ds(row0 + k * CP, CP)],
                    osem).start()

            @pl.when(cid == 1)
            def _():
                pltpu.make_async_copy(
                    vbufs[p], acc1.at[pl.ds(row0 + k * CP, CP)],
                    osem).start()
        for k in (NCP - 2, NCP - 1):
            out_desc(k, k % 2).wait()

    return pl.kernel(body, out_type=out_type, mesh=mesh,
                     scratch_types=scratch,
                     compiler_params=pltpu.CompilerParams(
                         use_tc_tiling_on_sc=False),
                     name="sc_segsum")


def _sc_deg():
    out_type = [jax.ShapeDtypeStruct((N_PAD, 16), jnp.float32),
                jax.ShapeDtypeStruct((N_PAD, 16), jnp.float32)]
    scratch = [
        pltpu.VMEM_SHARED((N_PAD, 16), jnp.float32),
        pltpu.VMEM((E_PER,), jnp.int32),
        pltpu.VMEM((CH, 16), jnp.float32),
        pltpu.VMEM((DP, 16), jnp.float32),
        pltpu.SemaphoreType.DMA,
        pltpu.SemaphoreType.DMA,
    ]
    mesh = plsc.VectorSubcoreMesh(core_axis_name="c", subcore_axis_name="s",
                                  num_cores=NC, num_subcores=NS)

    def body(eidx, zdeg, ones, deg0, deg1, deg_sh, dstbuf, ones_v, dbuf,
             gsem, dsem):
        cid = lax.axis_index("c")
        sid = lax.axis_index("s")
        wid = cid * NS + sid
        row0 = sid * ROWS_PER_TILE

        ih = pltpu.async_copy(eidx.at[1, pl.ds(wid * E_PER, E_PER)], dstbuf,
                              gsem)
        pltpu.sync_copy(zdeg, dbuf)
        zh = [pltpu.async_copy(dbuf, deg_sh.at[pl.ds(row0 + k * DP, DP)],
                               dsem)
              for k in range(NDP)]
        pltpu.sync_copy(ones, ones_v)
        for h in zh:
            h.wait()
        ih.wait()
        plsc.subcore_barrier()

        def d_desc(i):
            return pltpu.make_async_copy(
                ones_v, deg_sh.at[dstbuf.at[pl.ds(i * CH, CH)]], dsem)

        def step(i, _):
            d_desc(i).start(add=True)

            @pl.when(i >= 4)
            def _():
                d_desc(i - 4).wait()
            return 0

        lax.fori_loop(0, ITERS, step, 0)
        for i in range(ITERS - 4, ITERS):
            d_desc(i).wait()
        plsc.subcore_barrier()

        for k in range(NDP):
            r0 = row0 + k * DP
            pltpu.sync_copy(deg_sh.at[pl.ds(r0, DP)], dbuf)

            @pl.when(cid == 0)
            def _():
                pltpu.sync_copy(dbuf, deg0.at[pl.ds(r0, DP)])

            @pl.when(cid == 1)
            def _():
                pltpu.sync_copy(dbuf, deg1.at[pl.ds(r0, DP)])

    return pl.kernel(body, out_type=out_type, mesh=mesh,
                     scratch_types=scratch,
                     compiler_params=pltpu.CompilerParams(
                         use_tc_tiling_on_sc=False),
                     name="sc_deg")


_sc_segsum_k = _sc_segsum()
_sc_deg_k = _sc_deg()


BR = 400
GRID = N // BR


def _k1_body(x_ref, ws_ref, wn_ref, s_ref, p_ref):
    xb = x_ref[...]
    s_ref[...] = jnp.dot(xb, ws_ref[...], preferred_element_type=jnp.float32)
    p_ref[...] = jnp.dot(xb, wn_ref[...], preferred_element_type=jnp.float32)


def _tc_project(x, w_self, w_neigh):
    return pl.pallas_call(
        _k1_body,
        grid=(GRID,),
        in_specs=[
            pl.BlockSpec((BR, D_IN), lambda i: (i, 0)),
            pl.BlockSpec((D_IN, D_HID), lambda i: (0, 0)),
            pl.BlockSpec((D_IN, D_HID), lambda i: (0, 0)),
        ],
        out_specs=[
            pl.BlockSpec((BR, D_HID), lambda i: (i, 0)),
            pl.BlockSpec((BR, D_HID), lambda i: (i, 0)),
        ],
        out_shape=[
            jax.ShapeDtypeStruct((N, D_HID), jnp.float32),
            jax.ShapeDtypeStruct((N, D_HID), jnp.float32),
        ],
    )(x, w_self, w_neigh)


def _k2_body(s_ref, a0_ref, a1_ref, d0_ref, d1_ref, b_ref, h_ref):
    deg = d0_ref[...][:, :1] + d1_ref[...][:, :1]
    inv = 1.0 / jnp.maximum(deg, 1.0)
    mean = (a0_ref[...] + a1_ref[...]) * inv
    h_ref[...] = jnp.maximum(s_ref[...] + mean + b_ref[...], 0.0)


def _tc_combine(s1, a0, a1, d0, d1, b1):
    return pl.pallas_call(
        _k2_body,
        grid=(GRID,),
        in_specs=[
            pl.BlockSpec((BR, D_HID), lambda i: (i, 0)),
            pl.BlockSpec((BR, D_HID), lambda i: (i, 0)),
            pl.BlockSpec((BR, D_HID), lambda i: (i, 0)),
            pl.BlockSpec((BR, 16), lambda i: (i, 0)),
            pl.BlockSpec((BR, 16), lambda i: (i, 0)),
            pl.BlockSpec((1, D_HID), lambda i: (0, 0)),
        ],
        out_specs=pl.BlockSpec((BR, D_HID), lambda i: (i, 0)),
        out_shape=jax.ShapeDtypeStruct((N, D_HID), jnp.float32),
    )(s1, a0, a1, d0, d1, b1)


def _k3_body(h_ref, a0_ref, a1_ref, d0_ref, d1_ref, ws_ref, wn_ref, b_ref,
             o_ref):
    deg = d0_ref[...][:, :1] + d1_ref[...][:, :1]
    inv = 1.0 / jnp.maximum(deg, 1.0)
    mean = (a0_ref[...] + a1_ref[...]) * inv
    o_ref[...] = (
        jnp.dot(h_ref[...], ws_ref[...], preferred_element_type=jnp.float32)
        + jnp.dot(mean, wn_ref[...], preferred_element_type=jnp.float32)
        + b_ref[...]
    )


def _tc_final(h, a0, a1, d0, d1, w_self, w_neigh, b2):
    return pl.pallas_call(
        _k3_body,
        grid=(GRID,),
        in_specs=[
            pl.BlockSpec((BR, D_HID), lambda i: (i, 0)),
            pl.BlockSpec((BR, D_HID), lambda i: (i, 0)),
            pl.BlockSpec((BR, D_HID), lambda i: (i, 0)),
            pl.BlockSpec((BR, 16), lambda i: (i, 0)),
            pl.BlockSpec((BR, 16), lambda i: (i, 0)),
            pl.BlockSpec((D_HID, D_OUT), lambda i: (0, 0)),
            pl.BlockSpec((D_HID, D_OUT), lambda i: (0, 0)),
            pl.BlockSpec((1, D_OUT), lambda i: (0, 0)),
        ],
        out_specs=pl.BlockSpec((BR, D_OUT), lambda i: (i, 0)),
        out_shape=jax.ShapeDtypeStruct((N, D_OUT), jnp.float32),
    )(h, a0, a1, d0, d1, w_self, w_neigh, b2)


@jax.jit
def kernel(x, edge_index, W1_self, W1_neigh, b1, W2_self, W2_neigh, b2):
    eidx = edge_index
    zrows = jnp.zeros((CP, D_HID), jnp.float32)
    zdeg = jnp.zeros((DP, 16), jnp.float32)
    ones = jnp.ones((CH, 16), jnp.float32)

    d0, d1 = _sc_deg_k(eidx, zdeg, ones)
    s1, p1 = _tc_project(x, W1_self, W1_neigh)
    a0, a1 = _sc_segsum_k(p1, eidx, zrows)
    h = _tc_combine(s1, a0, a1, d0, d1, b1.reshape(1, D_HID))
    ah0, ah1 = _sc_segsum_k(h, eidx, zrows)
    out = _tc_final(h, ah0, ah1, d0, d1, W2_self, W2_neigh,
                    b2.reshape(1, D_OUT))
    return out


# speedup vs baseline: 1.3643x; 1.3643x over previous
import functools

import jax
import jax.numpy as jnp
from jax import lax
from jax.experimental import pallas as pl
from jax.experimental.pallas import tpu as pltpu
from jax.experimental.pallas import tpu_sc as plsc

N = 10000
E = 160000
D_IN = 256
D_HID = 128
D_OUT = 256

NC = 2
NS = 16
NW = NC * NS
E_PER = E // NW
CH = 100
ITERS = E_PER // CH
N_PAD = 10240
ROWS_PER_TILE = N_PAD // NS
CP = 40
NCP = ROWS_PER_TILE // CP
DP = 128
NDP = ROWS_PER_TILE // DP


def _sc_segsum():
    out_type = [jax.ShapeDtypeStruct((N_PAD, D_HID), jnp.float32),
                jax.ShapeDtypeStruct((N_PAD, D_HID), jnp.float32)]

    scratch = [
        pltpu.VMEM_SHARED((N_PAD, D_HID), jnp.float32),
        pltpu.VMEM((CP, D_HID), jnp.float32),
        pltpu.VMEM((CP, D_HID), jnp.float32),
        pltpu.VMEM((CH, D_HID), jnp.float32),
        pltpu.VMEM((CH, D_HID), jnp.float32),
        pltpu.VMEM((ITERS, CH), jnp.int32),
        pltpu.VMEM((ITERS, CH), jnp.int32),
        pltpu.SemaphoreType.DMA,
        pltpu.SemaphoreType.DMA,
        pltpu.SemaphoreType.DMA,
    ]

    mesh = plsc.VectorSubcoreMesh(core_axis_name="c", subcore_axis_name="s",
                                  num_cores=NC, num_subcores=NS)

    def body(tab, eidx, zrows, acc0, acc1, acc_sh, vbuf0, vbuf1,
             rows0, rows1, sbuf, dstbuf, gsem, ssem, osem):
        cid = lax.axis_index("c")
        sid = lax.axis_index("s")
        wid = cid * NS + sid
        rows = (rows0, rows1)
        vbufs = (vbuf0, vbuf1)
        row0 = sid * ROWS_PER_TILE

        ih1 = pltpu.async_copy(eidx.at[0, wid], sbuf, osem)
        ih2 = pltpu.async_copy(eidx.at[1, wid], dstbuf, osem)

        pltpu.sync_copy(zrows, vbuf0)
        zh = [pltpu.async_copy(vbuf0, acc_sh.at[pl.ds(row0 + k * CP, CP)],
                               gsem)
              for k in range(NCP)]
        for h in zh:
            h.wait()
        ih1.wait()
        ih2.wait()
        plsc.subcore_barrier()

        def gather_start(i, b):
            pltpu.async_copy(tab.at[sbuf.at[i]], rows[b], gsem)

        def gather_wait(i, b):
            pltpu.make_async_copy(tab.at[sbuf.at[i]], rows[b], gsem).wait()

        def process(i, b):
            gather_wait(i, b)

            @pl.when(i >= 1)
            def _():
                pltpu.make_async_copy(rows[1 - b],
                                      acc_sh.at[dstbuf.at[i - 1]],
                                      ssem).wait()

            @pl.when(i < ITERS - 1)
            def _():
                gather_start(i + 1, 1 - b)

            pltpu.make_async_copy(rows[b], acc_sh.at[dstbuf.at[i]],
                                  ssem).start(add=True)

        gather_start(0, 0)

        def body2(k, _):
            process(2 * k, 0)
            process(2 * k + 1, 1)
            return 0

        lax.fori_loop(0, ITERS // 2, body2, 0)
        for i in range(ITERS - ITERS % 2, ITERS):
            process(i, i % 2)
        pltpu.make_async_copy(rows[(ITERS - 1) % 2],
                              acc_sh.at[dstbuf.at[ITERS - 1]], ssem).wait()
        plsc.subcore_barrier()

        def out_desc(k, p):
            return pltpu.make_async_copy(
                vbufs[p], acc0.at[pl.ds(row0 + k * CP, CP)], osem)

        for k in range(NCP):
            p = k % 2
            if k >= 2:
                out_desc(k - 2, p).wait()
            pltpu.sync_copy(acc_sh.at[pl.ds(row0 + k * CP, CP)], vbufs[p])

            @pl.when(cid == 0)
            def _():
                pltpu.make_async_copy(
                    vbufs[p], acc0.at[pl.ds(row0 + k * CP, CP)],
                    osem).start()

            @pl.when(cid == 1)
            def _():
                pltpu.make_async_copy(
                    vbufs[p], acc1.at[pl.ds(row0 + k * CP, CP)],
                    osem).start()
        for k in (NCP - 2, NCP - 1):
            out_desc(k, k % 2).wait()

    return pl.kernel(body, out_type=out_type, mesh=mesh,
                     scratch_types=scratch,
                     compiler_params=pltpu.CompilerParams(
                         use_tc_tiling_on_sc=False),
                     name="sc_segsum")


def _sc_deg():
    out_type = [jax.ShapeDtypeStruct((N_PAD, 16), jnp.float32),
                jax.ShapeDtypeStruct((N_PAD, 16), jnp.float32)]
    scratch = [
        pltpu.VMEM_SHARED((N_PAD, 16), jnp.float32),
        pltpu.VMEM((ITERS, CH), jnp.int32),
        pltpu.VMEM((CH, 16), jnp.float32),
        pltpu.VMEM((DP, 16), jnp.float32),
        pltpu.SemaphoreType.DMA,
        pltpu.SemaphoreType.DMA,
    ]
    mesh = plsc.VectorSubcoreMesh(core_axis_name="c", subcore_axis_name="s",
                                  num_cores=NC, num_subcores=NS)

    def body(eidx, zdeg, ones, deg0, deg1, deg_sh, dstbuf, ones_v, dbuf,
             gsem, dsem):
        cid = lax.axis_index("c")
        sid = lax.axis_index("s")
        wid = cid * NS + sid
        row0 = sid * ROWS_PER_TILE

        ih = pltpu.async_copy(eidx.at[1, wid], dstbuf, gsem)
        pltpu.sync_copy(zdeg, dbuf)
        zh = [pltpu.async_copy(dbuf, deg_sh.at[pl.ds(row0 + k * DP, DP)],
                               dsem)
              for k in range(NDP)]
        pltpu.sync_copy(ones, ones_v)
        for h in zh:
            h.wait()
        ih.wait()
        plsc.subcore_barrier()

        def d_desc(i):
            return pltpu.make_async_copy(ones_v, deg_sh.at[dstbuf.at[i]],
                                         dsem)

        def step(i, _):
            d_desc(i).start(add=True)

            @pl.when(i >= 4)
            def _():
                d_desc(i - 4).wait()
            return 0

        lax.fori_loop(0, ITERS, step, 0)
        for i in range(ITERS - 4, ITERS):
            d_desc(i).wait()
        plsc.subcore_barrier()

        for k in range(NDP):
            r0 = row0 + k * DP
            pltpu.sync_copy(deg_sh.at[pl.ds(r0, DP)], dbuf)

            @pl.when(cid == 0)
            def _():
                pltpu.sync_copy(dbuf, deg0.at[pl.ds(r0, DP)])

            @pl.when(cid == 1)
            def _():
                pltpu.sync_copy(dbuf, deg1.at[pl.ds(r0, DP)])

    return pl.kernel(body, out_type=out_type, mesh=mesh,
                     scratch_types=scratch,
                     compiler_params=pltpu.CompilerParams(
                         use_tc_tiling_on_sc=False),
                     name="sc_deg")


_sc_segsum_k = _sc_segsum()
_sc_deg_k = _sc_deg()


BR = 400
GRID = N // BR


def _k1_body(x_ref, ws_ref, wn_ref, s_ref, p_ref):
    xb = x_ref[...]
    s_ref[...] = jnp.dot(xb, ws_ref[...], preferred_element_type=jnp.float32)
    p_ref[...] = jnp.dot(xb, wn_ref[...], preferred_element_type=jnp.float32)


def _tc_project(x, w_self, w_neigh):
    return pl.pallas_call(
        _k1_body,
        grid=(GRID,),
        in_specs=[
            pl.BlockSpec((BR, D_IN), lambda i: (i, 0)),
            pl.BlockSpec((D_IN, D_HID), lambda i: (0, 0)),
            pl.BlockSpec((D_IN, D_HID), lambda i: (0, 0)),
        ],
        out_specs=[
            pl.BlockSpec((BR, D_HID), lambda i: (i, 0)),
            pl.BlockSpec((BR, D_HID), lambda i: (i, 0)),
        ],
        out_shape=[
            jax.ShapeDtypeStruct((N, D_HID), jnp.float32),
            jax.ShapeDtypeStruct((N, D_HID), jnp.float32),
        ],
    )(x, w_self, w_neigh)


def _k2_body(s_ref, a0_ref, a1_ref, d0_ref, d1_ref, b_ref, h_ref):
    deg = d0_ref[...][:, :1] + d1_ref[...][:, :1]
    inv = 1.0 / jnp.maximum(deg, 1.0)
    mean = (a0_ref[...] + a1_ref[...]) * inv
    h_ref[...] = jnp.maximum(s_ref[...] + mean + b_ref[...], 0.0)


def _tc_combine(s1, a0, a1, d0, d1, b1):
    return pl.pallas_call(
        _k2_body,
        grid=(GRID,),
        in_specs=[
            pl.BlockSpec((BR, D_HID), lambda i: (i, 0)),
            pl.BlockSpec((BR, D_HID), lambda i: (i, 0)),
            pl.BlockSpec((BR, D_HID), lambda i: (i, 0)),
            pl.BlockSpec((BR, 16), lambda i: (i, 0)),
            pl.BlockSpec((BR, 16), lambda i: (i, 0)),
            pl.BlockSpec((1, D_HID), lambda i: (0, 0)),
        ],
        out_specs=pl.BlockSpec((BR, D_HID), lambda i: (i, 0)),
        out_shape=jax.ShapeDtypeStruct((N, D_HID), jnp.float32),
    )(s1, a0, a1, d0, d1, b1)


def _k3_body(h_ref, a0_ref, a1_ref, d0_ref, d1_ref, ws_ref, wn_ref, b_ref,
             o_ref):
    deg = d0_ref[...][:, :1] + d1_ref[...][:, :1]
    inv = 1.0 / jnp.maximum(deg, 1.0)
    mean = (a0_ref[...] + a1_ref[...]) * inv
    o_ref[...] = (
        jnp.dot(h_ref[...], ws_ref[...], preferred_element_type=jnp.float32)
        + jnp.dot(mean, wn_ref[...], preferred_element_type=jnp.float32)
        + b_ref[...]
    )


def _tc_final(h, a0, a1, d0, d1, w_self, w_neigh, b2):
    return pl.pallas_call(
        _k3_body,
        grid=(GRID,),
        in_specs=[
            pl.BlockSpec((BR, D_HID), lambda i: (i, 0)),
            pl.BlockSpec((BR, D_HID), lambda i: (i, 0)),
            pl.BlockSpec((BR, D_HID), lambda i: (i, 0)),
            pl.BlockSpec((BR, 16), lambda i: (i, 0)),
            pl.BlockSpec((BR, 16), lambda i: (i, 0)),
            pl.BlockSpec((D_HID, D_OUT), lambda i: (0, 0)),
            pl.BlockSpec((D_HID, D_OUT), lambda i: (0, 0)),
            pl.BlockSpec((1, D_OUT), lambda i: (0, 0)),
        ],
        out_specs=pl.BlockSpec((BR, D_OUT), lambda i: (i, 0)),
        out_shape=jax.ShapeDtypeStruct((N, D_OUT), jnp.float32),
    )(h, a0, a1, d0, d1, w_self, w_neigh, b2)


@jax.jit
def kernel(x, edge_index, W1_self, W1_neigh, b1, W2_self, W2_neigh, b2):
    eidx = edge_index.reshape(2, NW, ITERS, CH)
    zrows = jnp.zeros((CP, D_HID), jnp.float32)
    zdeg = jnp.zeros((DP, 16), jnp.float32)
    ones = jnp.ones((CH, 16), jnp.float32)

    s1, p1 = _tc_project(x, W1_self, W1_neigh)
    d0, d1 = _sc_deg_k(eidx, zdeg, ones)
    a0, a1 = _sc_segsum_k(p1, eidx, zrows)
    h = _tc_combine(s1, a0, a1, d0, d1, b1.reshape(1, D_HID))
    ah0, ah1 = _sc_segsum_k(h, eidx, zrows)
    out = _tc_final(h, ah0, ah1, d0, d1, W2_self, W2_neigh,
                    b2.reshape(1, D_OUT))
    return out
